# baseline (speedup 1.0000x reference)
import numpy as np

import jax
import jax.numpy as jnp
from jax.experimental import pallas as pl
from jax.experimental.pallas import tpu as pltpu

_NPROJ = 50
_N = 256
_DIM = 3
_BATCH = 8


def _projections():
    z = jax.random.normal(jax.random.key(42), (_NPROJ, _DIM, 2), dtype=jnp.float32)
    q, _ = jnp.linalg.qr(z)
    return q


try:
    _PROJ_CONST = np.asarray(_projections())
except Exception:
    _PROJ_CONST = None


def _bitonic_sort_cols(x):
    n = x.shape[0]
    row = jax.lax.broadcasted_iota(jnp.int32, x.shape, 0)
    k = 2
    while k <= n:
        j = k // 2
        while j >= 1:
            upper = (row & j) != 0
            partner = jnp.where(upper, jnp.roll(x, j, axis=0),
                                jnp.roll(x, -j, axis=0))
            asc = (row & k) == 0
            take_min = asc != upper
            x = jnp.where(take_min, jnp.minimum(x, partner),
                          jnp.maximum(x, partner))
            j //= 2
        k *= 2
    return x


def _ssw_kernel(pt_ref, qt_ref, pab_ref, out_ref):
    pab = pab_ref[...]

    def coords(p):
        x = jax.lax.dot(p, pab, preferred_element_type=jnp.float32)
        return (jnp.pi + jnp.arctan2(-x[:, _NPROJ:], -x[:, :_NPROJ])) \
            * (1.0 / (2.0 * jnp.pi))

    ones = jnp.ones((_N, 1), jnp.float32)

    def shift_costs(ur, vb, shift, stride):
        circ = pltpu.roll(vb, shift, axis=1, stride=stride, stride_axis=0)
        diff = jnp.abs(ur - circ)
        d = jnp.minimum(diff, 1.0 - diff)
        return jax.lax.dot(d * d, ones,
                           preferred_element_type=jnp.float32)

    uvt = _bitonic_sort_cols(
        jnp.concatenate([coords(pt_ref[0]), coords(qt_ref[0])], axis=1))
    uv = uvt.T
    u = uv[:_NPROJ]
    v = uv[_NPROJ:]

    @pl.when(pl.program_id(0) == 0)
    def _init():
        out_ref[0] = jnp.zeros((1, 1), jnp.float32)

    total = jnp.zeros((1, 1), jnp.float32)
    for l in range(_NPROJ):
        vb = jnp.broadcast_to(v[l:l + 1, :], (_N, _N))
        fr = shift_costs(u[l:l + 1, :], vb, 0, 1)
        total = total + jnp.min(fr, axis=(0, 1), keepdims=True)
    out_ref[0] = out_ref[0] + jnp.sqrt(total * (1.0 / (_NPROJ * _N)))


def kernel(P_batch, Q_batch):
    proj = _projections() if _PROJ_CONST is None else jnp.asarray(_PROJ_CONST)
    pa = proj[:, :, 0]
    pb = proj[:, :, 1]
    pab = jnp.concatenate([pa, pb], axis=0).T

    losses = pl.pallas_call(
        _ssw_kernel,
        grid=(_BATCH,),
        in_specs=[
            pl.BlockSpec((1, _N, _DIM), lambda b: (b, 0, 0)),
            pl.BlockSpec((1, _N, _DIM), lambda b: (b, 0, 0)),
            pl.BlockSpec((_DIM, 2 * _NPROJ), lambda b: (0, 0)),
        ],
        out_specs=pl.BlockSpec((1, 1, 1), lambda b: (0, 0, 0)),
        out_shape=jax.ShapeDtypeStruct((1, 1, 1), jnp.float32),
        compiler_params=pltpu.CompilerParams(
            dimension_semantics=("arbitrary",)),
    )(P_batch, Q_batch, pab)
    return losses[0, 0, 0]


# speedup vs baseline: 1.6328x; 1.6328x over previous
import numpy as np

import jax
import jax.numpy as jnp
from jax.experimental import pallas as pl
from jax.experimental.pallas import tpu as pltpu

_NPROJ = 50
_N = 256
_DIM = 3
_BATCH = 8


def _projections():
    z = jax.random.normal(jax.random.key(42), (_NPROJ, _DIM, 2), dtype=jnp.float32)
    q, _ = jnp.linalg.qr(z)
    return q


try:
    _PROJ_CONST = np.asarray(_projections())
except Exception:
    _PROJ_CONST = None


def _bitonic_sort_cols(x):
    n = x.shape[0]
    row = jax.lax.broadcasted_iota(jnp.int32, x.shape, 0)
    k = 2
    while k <= n:
        j = k // 2
        while j >= 1:
            upper = (row & j) != 0
            partner = jnp.where(upper, jnp.roll(x, j, axis=0),
                                jnp.roll(x, -j, axis=0))
            asc = (row & k) == 0
            take_min = asc != upper
            x = jnp.where(take_min, jnp.minimum(x, partner),
                          jnp.maximum(x, partner))
            j //= 2
        k *= 2
    return x


def _ssw_kernel(pt_ref, qt_ref, pab_ref, out_ref):
    pab = pab_ref[...]

    def coords(p):
        x = jax.lax.dot(p, pab, preferred_element_type=jnp.float32)
        return (jnp.pi + jnp.arctan2(-x[:, _NPROJ:], -x[:, :_NPROJ])) \
            * (1.0 / (2.0 * jnp.pi))

    ones = jnp.ones((_N, 1), jnp.float32)

    def shift_costs(ur, vb, shift, stride):
        circ = pltpu.roll(vb, shift, axis=1, stride=stride, stride_axis=0)
        diff = jnp.abs(ur - circ)
        d = jnp.minimum(diff, 1.0 - diff)
        return jax.lax.dot(d * d, ones,
                           preferred_element_type=jnp.float32)

    uvt = _bitonic_sort_cols(
        jnp.concatenate([coords(pt_ref[0]), coords(qt_ref[0])], axis=1))
    uv = uvt.T
    u = uv[:_NPROJ]
    v = uv[_NPROJ:]

    total = jnp.zeros((1, 1), jnp.float32)
    for l in range(_NPROJ):
        vb = jnp.broadcast_to(v[l:l + 1, :], (_N, _N))
        fr = shift_costs(u[l:l + 1, :], vb, 0, 1)
        total = total + jnp.min(fr, axis=(0, 1), keepdims=True)
    out_ref[0] = jnp.sqrt(total * (1.0 / (_NPROJ * _N)))


def kernel(P_batch, Q_batch):
    proj = _projections() if _PROJ_CONST is None else jnp.asarray(_PROJ_CONST)
    pa = proj[:, :, 0]
    pb = proj[:, :, 1]
    pab = jnp.concatenate([pa, pb], axis=0).T

    losses = pl.pallas_call(
        _ssw_kernel,
        grid=(_BATCH,),
        in_specs=[
            pl.BlockSpec((1, _N, _DIM), lambda b: (b, 0, 0)),
            pl.BlockSpec((1, _N, _DIM), lambda b: (b, 0, 0)),
            pl.BlockSpec((_DIM, 2 * _NPROJ), lambda b: (0, 0)),
        ],
        out_specs=pl.BlockSpec((1, 1, 1), lambda b: (b, 0, 0)),
        out_shape=jax.ShapeDtypeStruct((_BATCH, 1, 1), jnp.float32),
        compiler_params=pltpu.CompilerParams(
            dimension_semantics=("parallel",)),
    )(P_batch, Q_batch, pab)
    return jnp.sum(losses)
